# Initial kernel scaffold; baseline (speedup 1.0000x reference)
import jax
import jax.numpy as jnp
from jax.experimental import pallas as pl


def kernel(x, edge_index, W1, b1, W2, b2):
    raise NotImplementedError("write your pallas kernel here")


# speedup vs baseline: 6.6152x; 6.6152x over previous
import functools

import jax
import jax.numpy as jnp
from jax import lax
from jax.experimental import pallas as pl
from jax.experimental.pallas import tpu as pltpu
from jax.experimental.pallas import tpu_sc as plsc

NC, NS, LANES = 2, 16, 16
NW = NC * NS
B = 128

N_PAD = 10240
TC_ROWS = 1280


def _mesh():
    return plsc.VectorSubcoreMesh(
        core_axis_name="c", subcore_axis_name="s",
        num_cores=NC, num_subcores=NS)


CH = 8


def _make_spmm(n_pad, f_half, nch):
    rpt = n_pad // NS

    @functools.partial(
        pl.kernel,
        out_type=pltpu.HBM((NC, n_pad, f_half), jnp.float32),
        mesh=_mesh(),
        scratch_types=[
            pltpu.VMEM((CH, 2, B), jnp.int32),
            pltpu.VMEM((2, B, f_half), jnp.float32),
            pltpu.VMEM_SHARED((n_pad, f_half), jnp.float32),
            pltpu.SemaphoreType.DMA,
            pltpu.SemaphoreType.DMA,
        ],
    )
    def spmm_kernel(hs_hbm, sd_hbm, out_hbm, idx_v, rows_v, acc_sh,
                    sem0, sem1):
        c = lax.axis_index("c")
        s = lax.axis_index("s")
        hs_c = hs_hbm.at[c]
        sems = (sem0, sem1)
        pltpu.sync_copy(hs_hbm.at[c, pl.ds(s * rpt, rpt)],
                        acc_sh.at[pl.ds(s * rpt, rpt)])
        plsc.subcore_barrier()

        def gather(b, sem):
            pltpu.async_copy(hs_c.at[idx_v.at[b, 0]], rows_v.at[b % 2], sem)

        def gwait(b, sem):
            pltpu.make_async_copy(hs_c.at[idx_v.at[b, 0]], rows_v.at[b % 2],
                                  sem).wait()

        def body(k, carry):
            pltpu.sync_copy(sd_hbm.at[s, k], idx_v)
            gather(0, sems[0])
            for b in range(CH):
                if b + 1 < CH:
                    gather(b + 1, sems[(b + 1) % 2])
                gwait(b, sems[b % 2])
                pltpu.sync_copy(rows_v.at[b % 2], acc_sh.at[idx_v.at[b, 1]],
                                add=True)
            return carry
        lax.fori_loop(0, nch, body, 0)

        plsc.subcore_barrier()
        pltpu.sync_copy(acc_sh.at[pl.ds(s * rpt, rpt)],
                        out_hbm.at[c, pl.ds(s * rpt, rpt)])

    return spmm_kernel


def _make_spmm_edge(n_pad, f, nch):
    rpt = n_pad // NS

    @functools.partial(
        pl.kernel,
        out_type=pltpu.HBM((NC, n_pad, f), jnp.float32),
        mesh=_mesh(),
        scratch_types=[
            pltpu.VMEM((CH, 2, B), jnp.int32),
            pltpu.VMEM((2, B, f), jnp.float32),
            pltpu.VMEM_SHARED((n_pad, f), jnp.float32),
            pltpu.SemaphoreType.DMA,
            pltpu.SemaphoreType.DMA,
        ],
    )
    def spmm_kernel(hs_hbm, sd_hbm, out_hbm, idx_v, rows_v, acc_sh,
                    sem0, sem1):
        c = lax.axis_index("c")
        s = lax.axis_index("s")
        w = c * NS + s
        sems = (sem0, sem1)
        pltpu.sync_copy(hs_hbm.at[pl.ds(s * rpt, rpt)],
                        acc_sh.at[pl.ds(s * rpt, rpt)])
        plsc.subcore_barrier()

        def gather(b, sem):
            pltpu.async_copy(hs_hbm.at[idx_v.at[b, 0]], rows_v.at[b % 2], sem)

        def gwait(b, sem):
            pltpu.make_async_copy(hs_hbm.at[idx_v.at[b, 0]],
                                  rows_v.at[b % 2], sem).wait()

        def body(k, carry):
            pltpu.sync_copy(sd_hbm.at[w, k], idx_v)
            gather(0, sems[0])
            for b in range(CH):
                if b + 1 < CH:
                    gather(b + 1, sems[(b + 1) % 2])
                gwait(b, sems[b % 2])
                pltpu.sync_copy(rows_v.at[b % 2], acc_sh.at[idx_v.at[b, 1]],
                                add=True)
            return carry
        lax.fori_loop(0, nch, body, 0)

        plsc.subcore_barrier()
        pltpu.sync_copy(acc_sh.at[pl.ds(s * rpt, rpt)],
                        out_hbm.at[c, pl.ds(s * rpt, rpt)])

    return spmm_kernel


def _dinv_block(degp_blk):
    deg = degp_blk[0, :, 0:1] + degp_blk[1, :, 0:1] - 1.0
    return lax.rsqrt(jnp.maximum(deg, 1.0))


def _tc_a_body(x_ref, w_ref, degp_ref, out_ref):
    dinv = _dinv_block(degp_ref[...])
    h = jnp.dot(x_ref[...], w_ref[...], preferred_element_type=jnp.float32)
    hs = h * dinv
    f = h.shape[1] // 2
    out_ref[0] = hs[:, :f]
    out_ref[1] = hs[:, f:]


def _tc_b_body(acc_ref, degp_ref, b1_ref, w2_ref, out_ref):
    dinv = _dinv_block(degp_ref[...])
    accf = jnp.concatenate([acc_ref[0], acc_ref[1]], axis=1)
    h1 = jnp.maximum(accf * dinv + b1_ref[...], 0.0)
    out_ref[...] = jnp.dot(h1, w2_ref[...],
                           preferred_element_type=jnp.float32) * dinv


def _tc_c_body(acc_ref, hs2_ref, degp_ref, b2_ref, out_ref):
    dinv = _dinv_block(degp_ref[...])
    accf = acc_ref[0] + acc_ref[1] - hs2_ref[...]
    out_ref[...] = jnp.maximum(accf * dinv + b2_ref[...], 0.0)


def _row_spec(r, width):
    return pl.BlockSpec((r, width), lambda i: (i, 0))


def _half_spec(r, half):
    return pl.BlockSpec((2, r, half), lambda i: (0, i, 0))


def _degp_spec(r):
    return pl.BlockSpec((2, r, 128), lambda i: (0, i, 0))


def _full_spec(shape):
    return pl.BlockSpec(shape, lambda i: (0,) * len(shape))


def kernel(x, edge_index, W1, b1, W2, b2):
    n, d_in = x.shape
    d_hid = W1.shape[1]
    d_out = W2.shape[1]
    e = edge_index.shape[1]

    chunk = NW * B * CH
    e_pad = -(-e // chunk) * chunk
    nch = e_pad // (NS * B * CH)
    nch2 = e_pad // (NW * B * CH)

    src = edge_index[0].astype(jnp.int32)
    dst = edge_index[1].astype(jnp.int32)
    pad = e_pad - e
    dummy = jnp.full((pad,), n, dtype=jnp.int32)
    src_p = jnp.concatenate([src, dummy])
    dst_p = jnp.concatenate([dst, dummy])
    sd5 = jnp.stack([src_p.reshape(NS, nch, CH, B),
                     dst_p.reshape(NS, nch, CH, B)], axis=3)
    sdw5 = jnp.stack([src_p.reshape(NW, nch2, CH, B),
                      dst_p.reshape(NW, nch2, CH, B)], axis=3)
    sdd5 = jnp.stack([jnp.bitwise_and(dst_p, 127).reshape(NW, nch2, CH, B),
                      dst_p.reshape(NW, nch2, CH, B)], axis=3)

    x_p = jnp.zeros((N_PAD, d_in), x.dtype).at[:n].set(x)
    ones = jnp.ones((N_PAD, 128), jnp.float32)
    b1r = b1.reshape(1, d_hid)
    b2r = b2.reshape(1, d_out)

    grid = (N_PAD // TC_ROWS,)
    r = TC_ROWS

    degp = _make_spmm_edge(N_PAD, 128, nch2)(ones, sdd5)

    hs1 = pl.pallas_call(
        _tc_a_body,
        grid=grid,
        in_specs=[_row_spec(r, d_in), _full_spec((d_in, d_hid)), _degp_spec(r)],
        out_specs=_half_spec(r, d_hid // 2),
        out_shape=jax.ShapeDtypeStruct((2, N_PAD, d_hid // 2), jnp.float32),
    )(x_p, W1, degp)

    acc1 = _make_spmm(N_PAD, d_hid // 2, nch)(hs1, sd5)

    hs2 = pl.pallas_call(
        _tc_b_body,
        grid=grid,
        in_specs=[_half_spec(r, d_hid // 2), _degp_spec(r),
                  _full_spec((1, d_hid)), _full_spec((d_hid, d_out))],
        out_specs=_row_spec(r, d_out),
        out_shape=jax.ShapeDtypeStruct((N_PAD, d_out), jnp.float32),
    )(acc1, degp, b1r, W2)

    acc2 = _make_spmm_edge(N_PAD, d_out, nch2)(hs2, sdw5)

    out = pl.pallas_call(
        _tc_c_body,
        grid=grid,
        in_specs=[_half_spec(r, d_out), _row_spec(r, d_out), _degp_spec(r),
                  _full_spec((1, d_out))],
        out_specs=_row_spec(r, d_out),
        out_shape=jax.ShapeDtypeStruct((N_PAD, d_out), jnp.float32),
    )(acc2, hs2, degp, b2r)

    return out[:n]


# speedup vs baseline: 8.7182x; 1.3179x over previous
import functools

import jax
import jax.numpy as jnp
from jax import lax
from jax.experimental import pallas as pl
from jax.experimental.pallas import tpu as pltpu
from jax.experimental.pallas import tpu_sc as plsc

NC, NS, LANES = 2, 16, 16
NW = NC * NS
B = 128

N_PAD = 10240
TC_ROWS = 1280


def _mesh():
    return plsc.VectorSubcoreMesh(
        core_axis_name="c", subcore_axis_name="s",
        num_cores=NC, num_subcores=NS)


def _make_deg_hist(n_pad, nv):
    nr = n_pad // 128
    nft = nr // 8

    @functools.partial(
        pl.kernel,
        out_type=pltpu.HBM((NC, nr, 128), jnp.float32),
        mesh=_mesh(),
        compiler_params=pltpu.CompilerParams(needs_layout_passes=False),
        scratch_types=[
            pltpu.VMEM((nv, 16), jnp.int32),
            pltpu.VMEM((nr, 128), jnp.float32),
            pltpu.VMEM((nr,), jnp.int32),
            pltpu.VMEM_SHARED((nr, 128), jnp.float32),
        ],
    )
    def deg_kernel(dst_hbm, iota_hbm, out_hbm, idx_v, deg_v, iota_v, deg_sh):
        c = lax.axis_index("c")
        s = lax.axis_index("s")
        w = c * NS + s
        pltpu.sync_copy(dst_hbm.at[w], idx_v)
        pltpu.sync_copy(iota_hbm, iota_v)
        zeros = jnp.zeros((16,), jnp.float32)

        def zbody(i, carry):
            for k in range(8):
                deg_v[i, pl.ds(k * 16, 16)] = zeros
            return carry
        lax.fori_loop(0, nr, zbody, 0)

        @pl.when(s < nft)
        def _():
            pltpu.sync_copy(deg_v.at[pl.ds(0, 8)],
                            deg_sh.at[pl.ds(s * 8, 8)])
        plsc.subcore_barrier()

        ones16 = jnp.ones((16,), jnp.float32)

        def body(i, carry):
            d16 = idx_v[i]
            hi = lax.shift_right_logical(d16, 7)
            lo = lax.bitwise_and(d16, 127)
            plsc.addupdate_scatter(deg_v, [hi, lo], ones16)
            return carry
        lax.fori_loop(0, nv, body, 0)

        pltpu.sync_copy(deg_v, deg_sh.at[iota_v], add=True)
        plsc.subcore_barrier()

        @pl.when(s < nft)
        def _():
            pltpu.sync_copy(deg_sh.at[pl.ds(s * 8, 8)],
                            out_hbm.at[c, pl.ds(s * 8, 8)])

    return deg_kernel


CH = 8


def _make_spmm(n_pad, f_half, nch):
    rpt = n_pad // NS

    @functools.partial(
        pl.kernel,
        out_type=pltpu.HBM((NC, n_pad, f_half), jnp.float32),
        mesh=_mesh(),
        scratch_types=[
            pltpu.VMEM((CH, 2, B), jnp.int32),
            pltpu.VMEM((2, B, f_half), jnp.float32),
            pltpu.VMEM_SHARED((n_pad, f_half), jnp.float32),
            pltpu.SemaphoreType.DMA,
            pltpu.SemaphoreType.DMA,
        ],
    )
    def spmm_kernel(hs_hbm, sd_hbm, out_hbm, idx_v, rows_v, acc_sh,
                    sem0, sem1):
        c = lax.axis_index("c")
        s = lax.axis_index("s")
        hs_c = hs_hbm.at[c]
        sems = (sem0, sem1)
        pltpu.sync_copy(hs_hbm.at[c, pl.ds(s * rpt, rpt)],
                        acc_sh.at[pl.ds(s * rpt, rpt)])
        plsc.subcore_barrier()

        def gather(b, sem):
            pltpu.async_copy(hs_c.at[idx_v.at[b, 0]], rows_v.at[b % 2], sem)

        def gwait(b, sem):
            pltpu.make_async_copy(hs_c.at[idx_v.at[b, 0]], rows_v.at[b % 2],
                                  sem).wait()

        def body(k, carry):
            pltpu.sync_copy(sd_hbm.at[s, k], idx_v)
            gather(0, sems[0])
            for b in range(CH):
                if b + 1 < CH:
                    gather(b + 1, sems[(b + 1) % 2])
                gwait(b, sems[b % 2])
                pltpu.sync_copy(rows_v.at[b % 2], acc_sh.at[idx_v.at[b, 1]],
                                add=True)
            return carry
        lax.fori_loop(0, nch, body, 0)

        plsc.subcore_barrier()
        pltpu.sync_copy(acc_sh.at[pl.ds(s * rpt, rpt)],
                        out_hbm.at[c, pl.ds(s * rpt, rpt)])

    return spmm_kernel


def _make_spmm_edge(n_pad, f, nch):
    rpt = n_pad // NS

    @functools.partial(
        pl.kernel,
        out_type=pltpu.HBM((NC, n_pad, f), jnp.float32),
        mesh=_mesh(),
        scratch_types=[
            pltpu.VMEM((CH, 2, B), jnp.int32),
            pltpu.VMEM((2, B, f), jnp.float32),
            pltpu.VMEM_SHARED((n_pad, f), jnp.float32),
            pltpu.SemaphoreType.DMA,
            pltpu.SemaphoreType.DMA,
        ],
    )
    def spmm_kernel(hs_hbm, sd_hbm, out_hbm, idx_v, rows_v, acc_sh,
                    sem0, sem1):
        c = lax.axis_index("c")
        s = lax.axis_index("s")
        w = c * NS + s
        sems = (sem0, sem1)
        pltpu.sync_copy(hs_hbm.at[pl.ds(s * rpt, rpt)],
                        acc_sh.at[pl.ds(s * rpt, rpt)])
        plsc.subcore_barrier()

        def gather(b, sem):
            pltpu.async_copy(hs_hbm.at[idx_v.at[b, 0]], rows_v.at[b % 2], sem)

        def gwait(b, sem):
            pltpu.make_async_copy(hs_hbm.at[idx_v.at[b, 0]],
                                  rows_v.at[b % 2], sem).wait()

        def body(k, carry):
            pltpu.sync_copy(sd_hbm.at[w, k], idx_v)
            gather(0, sems[0])
            for b in range(CH):
                if b + 1 < CH:
                    gather(b + 1, sems[(b + 1) % 2])
                gwait(b, sems[b % 2])
                pltpu.sync_copy(rows_v.at[b % 2], acc_sh.at[idx_v.at[b, 1]],
                                add=True)
            return carry
        lax.fori_loop(0, nch, body, 0)

        plsc.subcore_barrier()
        pltpu.sync_copy(acc_sh.at[pl.ds(s * rpt, rpt)],
                        out_hbm.at[c, pl.ds(s * rpt, rpt)])

    return spmm_kernel


def _dinv_block(degp_blk):
    deg = degp_blk[0, :, 0:1] + degp_blk[1, :, 0:1] + 1.0
    return lax.rsqrt(jnp.maximum(deg, 1.0))


def _tc_a_body(x_ref, w_ref, degp_ref, out_ref):
    dinv = _dinv_block(degp_ref[...])
    h = jnp.dot(x_ref[...], w_ref[...], preferred_element_type=jnp.float32)
    hs = h * dinv
    f = h.shape[1] // 2
    out_ref[0] = hs[:, :f]
    out_ref[1] = hs[:, f:]


def _tc_b_body(acc_ref, degp_ref, b1_ref, w2_ref, out_ref):
    dinv = _dinv_block(degp_ref[...])
    accf = jnp.concatenate([acc_ref[0], acc_ref[1]], axis=1)
    h1 = jnp.maximum(accf * dinv + b1_ref[...], 0.0)
    out_ref[...] = jnp.dot(h1, w2_ref[...],
                           preferred_element_type=jnp.float32) * dinv


def _tc_c_body(acc_ref, hs2_ref, degp_ref, b2_ref, out_ref):
    dinv = _dinv_block(degp_ref[...])
    accf = acc_ref[0] + acc_ref[1] - hs2_ref[...]
    out_ref[...] = jnp.maximum(accf * dinv + b2_ref[...], 0.0)


def _row_spec(r, width):
    return pl.BlockSpec((r, width), lambda i: (i, 0))


def _half_spec(r, half):
    return pl.BlockSpec((2, r, half), lambda i: (0, i, 0))


def _degp_spec(r):
    return pl.BlockSpec((2, r, 1), lambda i: (0, i, 0))


def _full_spec(shape):
    return pl.BlockSpec(shape, lambda i: (0,) * len(shape))


def kernel(x, edge_index, W1, b1, W2, b2):
    n, d_in = x.shape
    d_hid = W1.shape[1]
    d_out = W2.shape[1]
    e = edge_index.shape[1]

    chunk = NW * B * CH
    e_pad = -(-e // chunk) * chunk
    nch = e_pad // (NS * B * CH)
    nch2 = e_pad // (NW * B * CH)

    src = edge_index[0].astype(jnp.int32)
    dst = edge_index[1].astype(jnp.int32)
    pad = e_pad - e
    dummy = jnp.full((pad,), n, dtype=jnp.int32)
    src_p = jnp.concatenate([src, dummy])
    dst_p = jnp.concatenate([dst, dummy])
    sd5 = jnp.stack([src_p.reshape(NS, nch, CH, B),
                     dst_p.reshape(NS, nch, CH, B)], axis=3)
    sdw5 = jnp.stack([src_p.reshape(NW, nch2, CH, B),
                      dst_p.reshape(NW, nch2, CH, B)], axis=3)
    nv = e_pad // (NW * 16)
    dst16 = dst_p.reshape(NW, nv, 16)
    iota_nr = jnp.arange(N_PAD // 128, dtype=jnp.int32)

    x_p = jnp.zeros((N_PAD, d_in), x.dtype).at[:n].set(x)
    b1r = b1.reshape(1, d_hid)
    b2r = b2.reshape(1, d_out)

    grid = (N_PAD // TC_ROWS,)
    r = TC_ROWS

    degp = _make_deg_hist(N_PAD, nv)(dst16, iota_nr).reshape(NC, N_PAD, 1)

    hs1 = pl.pallas_call(
        _tc_a_body,
        grid=grid,
        in_specs=[_row_spec(r, d_in), _full_spec((d_in, d_hid)), _degp_spec(r)],
        out_specs=_half_spec(r, d_hid // 2),
        out_shape=jax.ShapeDtypeStruct((2, N_PAD, d_hid // 2), jnp.float32),
    )(x_p, W1, degp)

    acc1 = _make_spmm(N_PAD, d_hid // 2, nch)(hs1, sd5)

    hs2 = pl.pallas_call(
        _tc_b_body,
        grid=grid,
        in_specs=[_half_spec(r, d_hid // 2), _degp_spec(r),
                  _full_spec((1, d_hid)), _full_spec((d_hid, d_out))],
        out_specs=_row_spec(r, d_out),
        out_shape=jax.ShapeDtypeStruct((N_PAD, d_out), jnp.float32),
    )(acc1, degp, b1r, W2)

    acc2 = _make_spmm_edge(N_PAD, d_out, nch2)(hs2, sdw5)

    out = pl.pallas_call(
        _tc_c_body,
        grid=grid,
        in_specs=[_half_spec(r, d_out), _row_spec(r, d_out), _degp_spec(r),
                  _full_spec((1, d_out))],
        out_specs=_row_spec(r, d_out),
        out_shape=jax.ShapeDtypeStruct((N_PAD, d_out), jnp.float32),
    )(acc2, hs2, degp, b2r)

    return out[:n]


# speedup vs baseline: 9.0319x; 1.0360x over previous
import functools

import jax
import jax.numpy as jnp
from jax import lax
from jax.experimental import pallas as pl
from jax.experimental.pallas import tpu as pltpu
from jax.experimental.pallas import tpu_sc as plsc

NC, NS, LANES = 2, 16, 16
NW = NC * NS
B = 128

N_PAD = 10240
TC_ROWS = 1280


def _mesh():
    return plsc.VectorSubcoreMesh(
        core_axis_name="c", subcore_axis_name="s",
        num_cores=NC, num_subcores=NS)


def _make_deg_hist(n_pad, nv):
    nr = n_pad // 128
    nft = nr // 8

    @functools.partial(
        pl.kernel,
        out_type=pltpu.HBM((NC, nr, 128), jnp.float32),
        mesh=_mesh(),
        compiler_params=pltpu.CompilerParams(needs_layout_passes=False),
        scratch_types=[
            pltpu.VMEM((nv, 16), jnp.int32),
            pltpu.VMEM((nr, 128), jnp.float32),
            pltpu.VMEM((nr,), jnp.int32),
            pltpu.VMEM_SHARED((nr, 128), jnp.float32),
        ],
    )
    def deg_kernel(dst_hbm, iota_hbm, out_hbm, idx_v, deg_v, iota_v, deg_sh):
        c = lax.axis_index("c")
        s = lax.axis_index("s")
        w = c * NS + s
        pltpu.sync_copy(dst_hbm.at[w], idx_v)
        pltpu.sync_copy(iota_hbm, iota_v)
        zeros = jnp.zeros((16,), jnp.float32)

        def zbody(i, carry):
            for k in range(8):
                deg_v[i, pl.ds(k * 16, 16)] = zeros
            return carry
        lax.fori_loop(0, nr, zbody, 0)

        @pl.when(s < nft)
        def _():
            pltpu.sync_copy(deg_v.at[pl.ds(0, 8)],
                            deg_sh.at[pl.ds(s * 8, 8)])
        plsc.subcore_barrier()

        ones16 = jnp.ones((16,), jnp.float32)

        def body(i, carry):
            d16 = idx_v[i]
            hi = lax.shift_right_logical(d16, 7)
            lo = lax.bitwise_and(d16, 127)
            plsc.addupdate_scatter(deg_v, [hi, lo], ones16)
            return carry
        lax.fori_loop(0, nv, body, 0)

        pltpu.sync_copy(deg_v, deg_sh.at[iota_v], add=True)
        plsc.subcore_barrier()

        @pl.when(s < nft)
        def _():
            pltpu.sync_copy(deg_sh.at[pl.ds(s * 8, 8)],
                            out_hbm.at[c, pl.ds(s * 8, 8)])

    return deg_kernel


CH = 8


def _make_spmm(n_pad, f, nch, edge_split):
    rpt = n_pad // NS

    @functools.partial(
        pl.kernel,
        out_type=pltpu.HBM((NC, n_pad, f), jnp.float32),
        mesh=_mesh(),
        scratch_types=[
            pltpu.VMEM((2, CH, 2, B), jnp.int32),
            pltpu.VMEM((2, B, f), jnp.float32),
            pltpu.VMEM_SHARED((n_pad, f), jnp.float32),
            pltpu.SemaphoreType.DMA,
            pltpu.SemaphoreType.DMA,
            pltpu.SemaphoreType.DMA,
            pltpu.SemaphoreType.DMA,
            pltpu.SemaphoreType.DMA,
        ],
    )
    def spmm_kernel(hs_hbm, sd_hbm, out_hbm, idx_v, rows_v, acc_sh,
                    g0, g1, s0, s1, isem):
        c = lax.axis_index("c")
        s = lax.axis_index("s")
        hs_ref = hs_hbm if edge_split else hs_hbm.at[c]
        row = c * NS + s if edge_split else s
        gsem = (g0, g1)
        ssem = (s0, s1)

        pltpu.sync_copy(hs_ref.at[pl.ds(s * rpt, rpt)],
                        acc_sh.at[pl.ds(s * rpt, rpt)])
        pltpu.sync_copy(sd_hbm.at[row, 0], idx_v.at[0])
        plsc.subcore_barrier()

        def ga(q, b, p):
            return pltpu.make_async_copy(hs_ref.at[idx_v.at[q, b, 0]],
                                         rows_v.at[p], gsem[p])

        def sc(q, b, p):
            return pltpu.make_async_copy(rows_v.at[p],
                                         acc_sh.at[idx_v.at[q, b, 1]],
                                         ssem[p])

        def ichunk(k, q):
            return pltpu.make_async_copy(sd_hbm.at[row, k], idx_v.at[q],
                                         isem)

        ga(0, 0, 0).start()
        for b in range(CH):
            p = b % 2
            if b > 0:
                sc(0, b - 1, 1 - p).wait()
            if b == 0 and nch > 1:
                ichunk(1, 1).start()
            if b < CH - 1:
                ga(0, b + 1, 1 - p).start()
            elif nch > 1:
                ichunk(1, 1).wait()
                ga(1, 0, 1 - p).start()
            ga(0, b, p).wait()
            sc(0, b, p).start(add=True)

        def body(k, carry):
            q = lax.rem(k, 2)
            qp = 1 - q
            for b in range(CH):
                p = b % 2
                if b == 0:
                    sc(qp, CH - 1, 1 - p).wait()
                else:
                    sc(q, b - 1, 1 - p).wait()
                if b == 0:
                    @pl.when(k + 1 < nch)
                    def _():
                        ichunk(k + 1, qp).start()
                if b < CH - 1:
                    ga(q, b + 1, 1 - p).start()
                else:
                    @pl.when(k + 1 < nch)
                    def _():
                        ichunk(k + 1, qp).wait()
                        ga(qp, 0, 1 - p).start()
                ga(q, b, p).wait()
                sc(q, b, p).start(add=True)
            return carry
        lax.fori_loop(1, nch, body, 0)

        sc((nch - 1) % 2, CH - 1, (CH - 1) % 2).wait()
        plsc.subcore_barrier()
        pltpu.sync_copy(acc_sh.at[pl.ds(s * rpt, rpt)],
                        out_hbm.at[c, pl.ds(s * rpt, rpt)])

    return spmm_kernel


def _dinv_block(degp_blk):
    deg = degp_blk[0, :, 0:1] + degp_blk[1, :, 0:1] + 1.0
    return lax.rsqrt(jnp.maximum(deg, 1.0))


def _tc_a_body(x_ref, w_ref, degp_ref, out_ref):
    dinv = _dinv_block(degp_ref[...])
    h = jnp.dot(x_ref[...], w_ref[...], preferred_element_type=jnp.float32)
    hs = h * dinv
    f = h.shape[1] // 2
    out_ref[0] = hs[:, :f]
    out_ref[1] = hs[:, f:]


def _tc_b_body(acc_ref, degp_ref, b1_ref, w2_ref, out_ref):
    dinv = _dinv_block(degp_ref[...])
    accf = jnp.concatenate([acc_ref[0], acc_ref[1]], axis=1)
    h1 = jnp.maximum(accf * dinv + b1_ref[...], 0.0)
    out_ref[...] = jnp.dot(h1, w2_ref[...],
                           preferred_element_type=jnp.float32) * dinv


def _tc_c_body(acc_ref, hs2_ref, degp_ref, b2_ref, out_ref):
    dinv = _dinv_block(degp_ref[...])
    accf = acc_ref[0] + acc_ref[1] - hs2_ref[...]
    out_ref[...] = jnp.maximum(accf * dinv + b2_ref[...], 0.0)


def _row_spec(r, width):
    return pl.BlockSpec((r, width), lambda i: (i, 0))


def _half_spec(r, half):
    return pl.BlockSpec((2, r, half), lambda i: (0, i, 0))


def _degp_spec(r):
    return pl.BlockSpec((2, r, 1), lambda i: (0, i, 0))


def _full_spec(shape):
    return pl.BlockSpec(shape, lambda i: (0,) * len(shape))


def kernel(x, edge_index, W1, b1, W2, b2):
    n, d_in = x.shape
    d_hid = W1.shape[1]
    d_out = W2.shape[1]
    e = edge_index.shape[1]

    chunk = NW * B * CH
    e_pad = -(-e // chunk) * chunk
    nch = e_pad // (NS * B * CH)
    nch2 = e_pad // (NW * B * CH)

    src = edge_index[0].astype(jnp.int32)
    dst = edge_index[1].astype(jnp.int32)
    pad = e_pad - e
    dummy = jnp.full((pad,), n, dtype=jnp.int32)
    src_p = jnp.concatenate([src, dummy])
    dst_p = jnp.concatenate([dst, dummy])
    sd5 = jnp.stack([src_p.reshape(NS, nch, CH, B),
                     dst_p.reshape(NS, nch, CH, B)], axis=3)
    sdw5 = jnp.stack([src_p.reshape(NW, nch2, CH, B),
                      dst_p.reshape(NW, nch2, CH, B)], axis=3)
    nv = e_pad // (NW * 16)
    dst16 = dst_p.reshape(NW, nv, 16)
    iota_nr = jnp.arange(N_PAD // 128, dtype=jnp.int32)

    x_p = jnp.zeros((N_PAD, d_in), x.dtype).at[:n].set(x)
    b1r = b1.reshape(1, d_hid)
    b2r = b2.reshape(1, d_out)

    grid = (N_PAD // TC_ROWS,)
    r = TC_ROWS

    degp = _make_deg_hist(N_PAD, nv)(dst16, iota_nr).reshape(NC, N_PAD, 1)

    hs1 = pl.pallas_call(
        _tc_a_body,
        grid=grid,
        in_specs=[_row_spec(r, d_in), _full_spec((d_in, d_hid)), _degp_spec(r)],
        out_specs=_half_spec(r, d_hid // 2),
        out_shape=jax.ShapeDtypeStruct((2, N_PAD, d_hid // 2), jnp.float32),
    )(x_p, W1, degp)

    acc1 = _make_spmm(N_PAD, d_hid // 2, nch, False)(hs1, sd5)

    hs2 = pl.pallas_call(
        _tc_b_body,
        grid=grid,
        in_specs=[_half_spec(r, d_hid // 2), _degp_spec(r),
                  _full_spec((1, d_hid)), _full_spec((d_hid, d_out))],
        out_specs=_row_spec(r, d_out),
        out_shape=jax.ShapeDtypeStruct((N_PAD, d_out), jnp.float32),
    )(acc1, degp, b1r, W2)

    acc2 = _make_spmm(N_PAD, d_out, nch2, True)(hs2, sdw5)

    out = pl.pallas_call(
        _tc_c_body,
        grid=grid,
        in_specs=[_half_spec(r, d_out), _row_spec(r, d_out), _degp_spec(r),
                  _full_spec((1, d_out))],
        out_specs=_row_spec(r, d_out),
        out_shape=jax.ShapeDtypeStruct((N_PAD, d_out), jnp.float32),
    )(acc2, hs2, degp, b2r)

    return out[:n]


# speedup vs baseline: 27.8267x; 3.0809x over previous
import functools

import jax
import jax.numpy as jnp
from jax import lax
from jax.experimental import pallas as pl
from jax.experimental.pallas import tpu as pltpu
from jax.experimental.pallas import tpu_sc as plsc

NC, NS, LANES = 2, 16, 16
NW = NC * NS
B = 128

N_PAD = 10240
TC_ROWS = 1280


def _mesh():
    return plsc.VectorSubcoreMesh(
        core_axis_name="c", subcore_axis_name="s",
        num_cores=NC, num_subcores=NS)


def _make_deg_hist(n_pad, nv):
    nr = n_pad // 128
    nft = nr // 8

    @functools.partial(
        pl.kernel,
        out_type=pltpu.HBM((NC, nr, 128), jnp.float32),
        mesh=_mesh(),
        compiler_params=pltpu.CompilerParams(needs_layout_passes=False),
        scratch_types=[
            pltpu.VMEM((nv, 16), jnp.int32),
            pltpu.VMEM((nr, 128), jnp.float32),
            pltpu.VMEM((nr,), jnp.int32),
            pltpu.VMEM_SHARED((nr, 128), jnp.float32),
        ],
    )
    def deg_kernel(dst_hbm, iota_hbm, out_hbm, idx_v, deg_v, iota_v, deg_sh):
        c = lax.axis_index("c")
        s = lax.axis_index("s")
        w = c * NS + s
        pltpu.sync_copy(dst_hbm.at[w], idx_v)
        pltpu.sync_copy(iota_hbm, iota_v)
        zeros = jnp.zeros((16,), jnp.float32)

        def zbody(i, carry):
            for k in range(8):
                deg_v[i, pl.ds(k * 16, 16)] = zeros
            return carry
        lax.fori_loop(0, nr, zbody, 0)

        @pl.when(s < nft)
        def _():
            pltpu.sync_copy(deg_v.at[pl.ds(0, 8)],
                            deg_sh.at[pl.ds(s * 8, 8)])
        plsc.subcore_barrier()

        ones16 = jnp.ones((16,), jnp.float32)

        def body(i, carry):
            d16 = idx_v[i]
            hi = lax.shift_right_logical(d16, 7)
            lo = lax.bitwise_and(d16, 127)
            plsc.addupdate_scatter(deg_v, [hi, lo], ones16)
            return carry
        lax.fori_loop(0, nv, body, 0)

        pltpu.sync_copy(deg_v, deg_sh.at[iota_v], add=True)
        plsc.subcore_barrier()

        @pl.when(s < nft)
        def _():
            pltpu.sync_copy(deg_sh.at[pl.ds(s * 8, 8)],
                            out_hbm.at[c, pl.ds(s * 8, 8)])

    return deg_kernel


CH = 8


def _make_spmm(n_pad, f, nch, edge_split):
    rpt = n_pad // NS

    @functools.partial(
        pl.kernel,
        out_type=pltpu.HBM((NC, n_pad, f), jnp.float32),
        mesh=_mesh(),
        scratch_types=[
            pltpu.VMEM((2, CH, 2, B), jnp.int32),
            pltpu.VMEM((2, B, f), jnp.float32),
            pltpu.VMEM_SHARED((n_pad, f), jnp.float32),
            pltpu.SemaphoreType.DMA,
            pltpu.SemaphoreType.DMA,
            pltpu.SemaphoreType.DMA,
            pltpu.SemaphoreType.DMA,
            pltpu.SemaphoreType.DMA,
        ],
    )
    def spmm_kernel(hs_hbm, sd_hbm, out_hbm, idx_v, rows_v, acc_sh,
                    g0, g1, s0, s1, isem):
        c = lax.axis_index("c")
        s = lax.axis_index("s")
        hs_ref = hs_hbm if edge_split else hs_hbm.at[c]
        row = c * NS + s if edge_split else s
        gsem = (g0, g1)
        ssem = (s0, s1)

        pltpu.sync_copy(hs_ref.at[pl.ds(s * rpt, rpt)],
                        acc_sh.at[pl.ds(s * rpt, rpt)])
        pltpu.sync_copy(sd_hbm.at[row, 0], idx_v.at[0])
        plsc.subcore_barrier()

        def ga(q, b, p):
            return pltpu.make_async_copy(hs_ref.at[idx_v.at[q, b, 0]],
                                         rows_v.at[p], gsem[p])

        def sc(q, b, p):
            return pltpu.make_async_copy(rows_v.at[p],
                                         acc_sh.at[idx_v.at[q, b, 1]],
                                         ssem[p])

        def ichunk(k, q):
            return pltpu.make_async_copy(sd_hbm.at[row, k], idx_v.at[q],
                                         isem)

        ga(0, 0, 0).start()
        for b in range(CH):
            p = b % 2
            if b > 0:
                sc(0, b - 1, 1 - p).wait()
            if b == 0 and nch > 1:
                ichunk(1, 1).start()
            if b < CH - 1:
                ga(0, b + 1, 1 - p).start()
            elif nch > 1:
                ichunk(1, 1).wait()
                ga(1, 0, 1 - p).start()
            ga(0, b, p).wait()
            sc(0, b, p).start(add=True)

        def body(k, carry):
            q = lax.rem(k, 2)
            qp = 1 - q
            for b in range(CH):
                p = b % 2
                if b == 0:
                    sc(qp, CH - 1, 1 - p).wait()
                else:
                    sc(q, b - 1, 1 - p).wait()
                if b == 0:
                    @pl.when(k + 1 < nch)
                    def _():
                        ichunk(k + 1, qp).start()
                if b < CH - 1:
                    ga(q, b + 1, 1 - p).start()
                else:
                    @pl.when(k + 1 < nch)
                    def _():
                        ichunk(k + 1, qp).wait()
                        ga(qp, 0, 1 - p).start()
                ga(q, b, p).wait()
                sc(q, b, p).start(add=True)
            return carry
        lax.fori_loop(1, nch, body, 0)

        sc((nch - 1) % 2, CH - 1, (CH - 1) % 2).wait()
        plsc.subcore_barrier()
        pltpu.sync_copy(acc_sh.at[pl.ds(s * rpt, rpt)],
                        out_hbm.at[c, pl.ds(s * rpt, rpt)])

    return spmm_kernel


def _dinv_block(degp_blk):
    deg = degp_blk[0, :, 0:1] + degp_blk[1, :, 0:1] + 1.0
    return lax.rsqrt(jnp.maximum(deg, 1.0))


def _tc_a_body(x_ref, w_ref, degp_ref, out_ref):
    dinv = _dinv_block(degp_ref[...])
    h = jnp.dot(x_ref[...], w_ref[...], preferred_element_type=jnp.float32)
    hs = h * dinv
    f = h.shape[1] // 2
    out_ref[0] = hs[:, :f]
    out_ref[1] = hs[:, f:]


def _tc_b_body(acc_ref, degp_ref, b1_ref, w2_ref, out_ref):
    dinv = _dinv_block(degp_ref[...])
    accf = jnp.concatenate([acc_ref[0], acc_ref[1]], axis=1)
    h1 = jnp.maximum(accf * dinv + b1_ref[...], 0.0)
    out_ref[...] = jnp.dot(h1, w2_ref[...],
                           preferred_element_type=jnp.float32) * dinv


def _tc_c_body(acc_ref, hs2_ref, degp_ref, b2_ref, out_ref):
    dinv = _dinv_block(degp_ref[...])
    accf = acc_ref[0] + acc_ref[1] - hs2_ref[...]
    out_ref[...] = jnp.maximum(accf * dinv + b2_ref[...], 0.0)


def _row_spec(r, width):
    return pl.BlockSpec((r, width), lambda i: (i, 0))


def _half_spec(r, half):
    return pl.BlockSpec((2, r, half), lambda i: (0, i, 0))


def _degp_spec(r):
    return pl.BlockSpec((2, r, 1), lambda i: (0, i, 0))


def _full_spec(shape):
    return pl.BlockSpec(shape, lambda i: (0,) * len(shape))


def kernel(x, edge_index, W1, b1, W2, b2):
    n, d_in = x.shape
    d_hid = W1.shape[1]
    d_out = W2.shape[1]
    e = edge_index.shape[1]

    chunk = NW * B * CH
    e_pad = -(-e // chunk) * chunk
    nch = e_pad // (NS * B * CH)
    nch2 = e_pad // (NW * B * CH)

    src = edge_index[0].astype(jnp.int32)
    dst = edge_index[1].astype(jnp.int32)
    pad = e_pad - e
    npad_rows = N_PAD - n
    dummy = n + jnp.arange(pad, dtype=jnp.int32) % npad_rows
    src_p = jnp.concatenate([src, dummy])
    dst_p = jnp.concatenate([dst, dummy])
    sd5 = jnp.stack([src_p.reshape(NS, nch, CH, B),
                     dst_p.reshape(NS, nch, CH, B)], axis=3)
    sdw5 = jnp.stack([src_p.reshape(NW, nch2, CH, B),
                      dst_p.reshape(NW, nch2, CH, B)], axis=3)
    nv = e_pad // (NW * 16)
    dst16 = dst_p.reshape(NW, nv, 16)
    iota_nr = jnp.arange(N_PAD // 128, dtype=jnp.int32)

    x_p = jnp.zeros((N_PAD, d_in), x.dtype).at[:n].set(x)
    b1r = b1.reshape(1, d_hid)
    b2r = b2.reshape(1, d_out)

    grid = (N_PAD // TC_ROWS,)
    r = TC_ROWS

    degp = _make_deg_hist(N_PAD, nv)(dst16, iota_nr).reshape(NC, N_PAD, 1)

    hs1 = pl.pallas_call(
        _tc_a_body,
        grid=grid,
        in_specs=[_row_spec(r, d_in), _full_spec((d_in, d_hid)), _degp_spec(r)],
        out_specs=_half_spec(r, d_hid // 2),
        out_shape=jax.ShapeDtypeStruct((2, N_PAD, d_hid // 2), jnp.float32),
    )(x_p, W1, degp)

    acc1 = _make_spmm(N_PAD, d_hid // 2, nch, False)(hs1, sd5)

    hs2 = pl.pallas_call(
        _tc_b_body,
        grid=grid,
        in_specs=[_half_spec(r, d_hid // 2), _degp_spec(r),
                  _full_spec((1, d_hid)), _full_spec((d_hid, d_out))],
        out_specs=_row_spec(r, d_out),
        out_shape=jax.ShapeDtypeStruct((N_PAD, d_out), jnp.float32),
    )(acc1, degp, b1r, W2)

    acc2 = _make_spmm(N_PAD, d_out, nch2, True)(hs2, sdw5)

    out = pl.pallas_call(
        _tc_c_body,
        grid=grid,
        in_specs=[_half_spec(r, d_out), _row_spec(r, d_out), _degp_spec(r),
                  _full_spec((1, d_out))],
        out_specs=_row_spec(r, d_out),
        out_shape=jax.ShapeDtypeStruct((N_PAD, d_out), jnp.float32),
    )(acc2, hs2, degp, b2r)

    return out[:n]


# speedup vs baseline: 27.9173x; 1.0033x over previous
import functools

import jax
import jax.numpy as jnp
from jax import lax
from jax.experimental import pallas as pl
from jax.experimental.pallas import tpu as pltpu
from jax.experimental.pallas import tpu_sc as plsc

NC, NS, LANES = 2, 16, 16
NW = NC * NS
B = 128

N_PAD = 10240
TC_ROWS = 1280


def _mesh():
    return plsc.VectorSubcoreMesh(
        core_axis_name="c", subcore_axis_name="s",
        num_cores=NC, num_subcores=NS)


def _make_deg_hist(n_pad, nv):
    nr = n_pad // 128
    nft = nr // 8

    @functools.partial(
        pl.kernel,
        out_type=pltpu.HBM((NC, nr, 128), jnp.float32),
        mesh=_mesh(),
        compiler_params=pltpu.CompilerParams(needs_layout_passes=False,
                                             skip_device_barrier=True),
        scratch_types=[
            pltpu.VMEM((nv, 16), jnp.int32),
            pltpu.VMEM((nr, 128), jnp.float32),
            pltpu.VMEM((nr,), jnp.int32),
            pltpu.VMEM_SHARED((nr, 128), jnp.float32),
        ],
    )
    def deg_kernel(dst_hbm, iota_hbm, out_hbm, idx_v, deg_v, iota_v, deg_sh):
        c = lax.axis_index("c")
        s = lax.axis_index("s")
        w = c * NS + s
        pltpu.sync_copy(dst_hbm.at[w], idx_v)
        pltpu.sync_copy(iota_hbm, iota_v)
        zeros = jnp.zeros((16,), jnp.float32)

        def zbody(i, carry):
            for k in range(8):
                deg_v[i, pl.ds(k * 16, 16)] = zeros
            return carry
        lax.fori_loop(0, nr, zbody, 0)

        @pl.when(s < nft)
        def _():
            pltpu.sync_copy(deg_v.at[pl.ds(0, 8)],
                            deg_sh.at[pl.ds(s * 8, 8)])
        plsc.subcore_barrier()

        ones16 = jnp.ones((16,), jnp.float32)

        def body(i, carry):
            d16 = idx_v[i]
            hi = lax.shift_right_logical(d16, 7)
            lo = lax.bitwise_and(d16, 127)
            plsc.addupdate_scatter(deg_v, [hi, lo], ones16)
            return carry
        lax.fori_loop(0, nv, body, 0)

        pltpu.sync_copy(deg_v, deg_sh.at[iota_v], add=True)
        plsc.subcore_barrier()

        @pl.when(s < nft)
        def _():
            pltpu.sync_copy(deg_sh.at[pl.ds(s * 8, 8)],
                            out_hbm.at[c, pl.ds(s * 8, 8)])

    return deg_kernel


CH = 8


def _make_spmm(n_pad, f, nch, edge_split):
    rpt = n_pad // NS

    @functools.partial(
        pl.kernel,
        out_type=pltpu.HBM((NC, n_pad, f), jnp.float32),
        mesh=_mesh(),
        compiler_params=pltpu.CompilerParams(skip_device_barrier=True),
        scratch_types=[
            pltpu.VMEM((2, CH, 2, B), jnp.int32),
            pltpu.VMEM((2, B, f), jnp.float32),
            pltpu.VMEM_SHARED((n_pad, f), jnp.float32),
            pltpu.SemaphoreType.DMA,
            pltpu.SemaphoreType.DMA,
            pltpu.SemaphoreType.DMA,
            pltpu.SemaphoreType.DMA,
            pltpu.SemaphoreType.DMA,
        ],
    )
    def spmm_kernel(hs_hbm, sd_hbm, out_hbm, idx_v, rows_v, acc_sh,
                    g0, g1, s0, s1, isem):
        c = lax.axis_index("c")
        s = lax.axis_index("s")
        hs_ref = hs_hbm if edge_split else hs_hbm.at[c]
        row = c * NS + s if edge_split else s
        gsem = (g0, g1)
        ssem = (s0, s1)

        pltpu.sync_copy(hs_ref.at[pl.ds(s * rpt, rpt)],
                        acc_sh.at[pl.ds(s * rpt, rpt)])
        pltpu.sync_copy(sd_hbm.at[row, 0], idx_v.at[0])
        plsc.subcore_barrier()

        def ga(q, b, p):
            return pltpu.make_async_copy(hs_ref.at[idx_v.at[q, b, 0]],
                                         rows_v.at[p], gsem[p])

        def sc(q, b, p):
            return pltpu.make_async_copy(rows_v.at[p],
                                         acc_sh.at[idx_v.at[q, b, 1]],
                                         ssem[p])

        def ichunk(k, q):
            return pltpu.make_async_copy(sd_hbm.at[row, k], idx_v.at[q],
                                         isem)

        ga(0, 0, 0).start()
        for b in range(CH):
            p = b % 2
            if b > 0:
                sc(0, b - 1, 1 - p).wait()
            if b == 0 and nch > 1:
                ichunk(1, 1).start()
            if b < CH - 1:
                ga(0, b + 1, 1 - p).start()
            elif nch > 1:
                ichunk(1, 1).wait()
                ga(1, 0, 1 - p).start()
            ga(0, b, p).wait()
            sc(0, b, p).start(add=True)

        def body(k, carry):
            q = lax.rem(k, 2)
            qp = 1 - q
            for b in range(CH):
                p = b % 2
                if b == 0:
                    sc(qp, CH - 1, 1 - p).wait()
                else:
                    sc(q, b - 1, 1 - p).wait()
                if b == 0:
                    @pl.when(k + 1 < nch)
                    def _():
                        ichunk(k + 1, qp).start()
                if b < CH - 1:
                    ga(q, b + 1, 1 - p).start()
                else:
                    @pl.when(k + 1 < nch)
                    def _():
                        ichunk(k + 1, qp).wait()
                        ga(qp, 0, 1 - p).start()
                ga(q, b, p).wait()
                sc(q, b, p).start(add=True)
            return carry
        lax.fori_loop(1, nch, body, 0)

        sc((nch - 1) % 2, CH - 1, (CH - 1) % 2).wait()
        plsc.subcore_barrier()
        pltpu.sync_copy(acc_sh.at[pl.ds(s * rpt, rpt)],
                        out_hbm.at[c, pl.ds(s * rpt, rpt)])

    return spmm_kernel


def _dinv_block(degp_blk):
    deg = degp_blk[0, :, 0:1] + degp_blk[1, :, 0:1] + 1.0
    return lax.rsqrt(jnp.maximum(deg, 1.0))


def _tc_a_body(x_ref, w_ref, degp_ref, out_ref):
    dinv = _dinv_block(degp_ref[...])
    h = jnp.dot(x_ref[...], w_ref[...], preferred_element_type=jnp.float32)
    hs = h * dinv
    f = h.shape[1] // 2
    out_ref[0] = hs[:, :f]
    out_ref[1] = hs[:, f:]


def _tc_b_body(acc_ref, degp_ref, b1_ref, w2_ref, out_ref):
    dinv = _dinv_block(degp_ref[...])
    accf = jnp.concatenate([acc_ref[0], acc_ref[1]], axis=1)
    h1 = jnp.maximum(accf * dinv + b1_ref[...], 0.0)
    out_ref[...] = jnp.dot(h1, w2_ref[...],
                           preferred_element_type=jnp.float32) * dinv


def _tc_c_body(acc_ref, hs2_ref, degp_ref, b2_ref, out_ref):
    dinv = _dinv_block(degp_ref[...])
    accf = acc_ref[0] + acc_ref[1] - hs2_ref[...]
    out_ref[...] = jnp.maximum(accf * dinv + b2_ref[...], 0.0)


def _row_spec(r, width):
    return pl.BlockSpec((r, width), lambda i: (i, 0))


def _half_spec(r, half):
    return pl.BlockSpec((2, r, half), lambda i: (0, i, 0))


def _degp_spec(r):
    return pl.BlockSpec((2, r, 1), lambda i: (0, i, 0))


def _full_spec(shape):
    return pl.BlockSpec(shape, lambda i: (0,) * len(shape))


def kernel(x, edge_index, W1, b1, W2, b2):
    n, d_in = x.shape
    d_hid = W1.shape[1]
    d_out = W2.shape[1]
    e = edge_index.shape[1]

    chunk = NW * B * CH
    e_pad = -(-e // chunk) * chunk
    nch = e_pad // (NS * B * CH)
    nch2 = e_pad // (NW * B * CH)

    src = edge_index[0].astype(jnp.int32)
    dst = edge_index[1].astype(jnp.int32)
    pad = e_pad - e
    npad_rows = N_PAD - n
    dummy = n + jnp.arange(pad, dtype=jnp.int32) % npad_rows
    src_p = jnp.concatenate([src, dummy])
    dst_p = jnp.concatenate([dst, dummy])
    chunks = e_pad // (CH * B)
    sd = jnp.stack([src_p.reshape(chunks, CH, B),
                    dst_p.reshape(chunks, CH, B)], axis=2)
    sd5 = sd.reshape(NS, nch, CH, 2, B)
    sdw5 = sd.reshape(NW, nch2, CH, 2, B)
    nv = e_pad // (NW * 16)
    dst16 = dst_p.reshape(NW, nv, 16)
    iota_nr = jnp.arange(N_PAD // 128, dtype=jnp.int32)

    x_p = jnp.zeros((N_PAD, d_in), x.dtype).at[:n].set(x)
    b1r = b1.reshape(1, d_hid)
    b2r = b2.reshape(1, d_out)

    grid = (N_PAD // TC_ROWS,)
    r = TC_ROWS

    degp = _make_deg_hist(N_PAD, nv)(dst16, iota_nr).reshape(NC, N_PAD, 1)

    hs1 = pl.pallas_call(
        _tc_a_body,
        grid=grid,
        in_specs=[_row_spec(r, d_in), _full_spec((d_in, d_hid)), _degp_spec(r)],
        out_specs=_half_spec(r, d_hid // 2),
        out_shape=jax.ShapeDtypeStruct((2, N_PAD, d_hid // 2), jnp.float32),
    )(x_p, W1, degp)

    acc1 = _make_spmm(N_PAD, d_hid // 2, nch, False)(hs1, sd5)

    hs2 = pl.pallas_call(
        _tc_b_body,
        grid=grid,
        in_specs=[_half_spec(r, d_hid // 2), _degp_spec(r),
                  _full_spec((1, d_hid)), _full_spec((d_hid, d_out))],
        out_specs=_row_spec(r, d_out),
        out_shape=jax.ShapeDtypeStruct((N_PAD, d_out), jnp.float32),
    )(acc1, degp, b1r, W2)

    acc2 = _make_spmm(N_PAD, d_out, nch2, True)(hs2, sdw5)

    out = pl.pallas_call(
        _tc_c_body,
        grid=grid,
        in_specs=[_half_spec(r, d_out), _row_spec(r, d_out), _degp_spec(r),
                  _full_spec((1, d_out))],
        out_specs=_row_spec(r, d_out),
        out_shape=jax.ShapeDtypeStruct((N_PAD, d_out), jnp.float32),
    )(acc2, hs2, degp, b2r)

    return out[:n]


# speedup vs baseline: 28.3679x; 1.0161x over previous
import functools

import jax
import jax.numpy as jnp
from jax import lax
from jax.experimental import pallas as pl
from jax.experimental.pallas import tpu as pltpu
from jax.experimental.pallas import tpu_sc as plsc

NC, NS, LANES = 2, 16, 16
NW = NC * NS
B = 128

N_PAD = 10240
TC_ROWS = 1280


def _mesh():
    return plsc.VectorSubcoreMesh(
        core_axis_name="c", subcore_axis_name="s",
        num_cores=NC, num_subcores=NS)


def _make_deg_hist(n_pad, nv):
    nr = n_pad // 128
    nft = nr // 8

    @functools.partial(
        pl.kernel,
        out_type=pltpu.HBM((NC, nr, 128), jnp.float32),
        mesh=_mesh(),
        compiler_params=pltpu.CompilerParams(needs_layout_passes=False,
                                             skip_device_barrier=True),
        scratch_types=[
            pltpu.VMEM((nv, 16), jnp.int32),
            pltpu.VMEM((nr, 128), jnp.float32),
            pltpu.VMEM((nr,), jnp.int32),
            pltpu.VMEM_SHARED((nr, 128), jnp.float32),
        ],
    )
    def deg_kernel(dst_hbm, iota_hbm, out_hbm, idx_v, deg_v, iota_v, deg_sh):
        c = lax.axis_index("c")
        s = lax.axis_index("s")
        w = c * NS + s
        pltpu.sync_copy(dst_hbm.at[w], idx_v)
        pltpu.sync_copy(iota_hbm, iota_v)
        zeros = jnp.zeros((16,), jnp.float32)

        def zbody(i, carry):
            for k in range(8):
                deg_v[i, pl.ds(k * 16, 16)] = zeros
            return carry
        lax.fori_loop(0, nr, zbody, 0)

        @pl.when(s < nft)
        def _():
            pltpu.sync_copy(deg_v.at[pl.ds(0, 8)],
                            deg_sh.at[pl.ds(s * 8, 8)])
        plsc.subcore_barrier()

        ones16 = jnp.ones((16,), jnp.float32)

        def body(i, carry):
            d16 = idx_v[i]
            hi = lax.shift_right_logical(d16, 7)
            lo = lax.bitwise_and(d16, 127)
            plsc.addupdate_scatter(deg_v, [hi, lo], ones16)
            return carry
        lax.fori_loop(0, nv, body, 0)

        pltpu.sync_copy(deg_v, deg_sh.at[iota_v], add=True)
        plsc.subcore_barrier()

        @pl.when(s < nft)
        def _():
            pltpu.sync_copy(deg_sh.at[pl.ds(s * 8, 8)],
                            out_hbm.at[c, pl.ds(s * 8, 8)])

    return deg_kernel


CH = 8


def _make_spmm(n_pad, f, nch, edge_split):
    rpt = n_pad // NS

    @functools.partial(
        pl.kernel,
        out_type=pltpu.HBM((NC, n_pad, f), jnp.float32),
        mesh=_mesh(),
        compiler_params=pltpu.CompilerParams(skip_device_barrier=True),
        scratch_types=[
            pltpu.VMEM((2, CH, 2, B), jnp.int32),
            pltpu.VMEM((2, B, f), jnp.float32),
            pltpu.VMEM_SHARED((n_pad, f), jnp.float32),
            pltpu.SemaphoreType.DMA,
            pltpu.SemaphoreType.DMA,
            pltpu.SemaphoreType.DMA,
            pltpu.SemaphoreType.DMA,
            pltpu.SemaphoreType.DMA,
        ],
    )
    def spmm_kernel(hs_hbm, sd_hbm, out_hbm, idx_v, rows_v, acc_sh,
                    g0, g1, s0, s1, isem):
        c = lax.axis_index("c")
        s = lax.axis_index("s")
        hs_ref = hs_hbm if edge_split else hs_hbm.at[c]
        row = c * NS + s if edge_split else s
        gsem = (g0, g1)
        ssem = (s0, s1)

        pltpu.sync_copy(hs_ref.at[pl.ds(s * rpt, rpt)],
                        acc_sh.at[pl.ds(s * rpt, rpt)])
        pltpu.sync_copy(sd_hbm.at[row, 0], idx_v.at[0])
        plsc.subcore_barrier()

        def ga(q, b, p):
            return pltpu.make_async_copy(hs_ref.at[idx_v.at[q, b, 0]],
                                         rows_v.at[p], gsem[p])

        def sc(q, b, p):
            return pltpu.make_async_copy(rows_v.at[p],
                                         acc_sh.at[idx_v.at[q, b, 1]],
                                         ssem[p])

        def ichunk(k, q):
            return pltpu.make_async_copy(sd_hbm.at[row, k], idx_v.at[q],
                                         isem)

        ga(0, 0, 0).start()
        for b in range(CH):
            p = b % 2
            if b > 0:
                sc(0, b - 1, 1 - p).wait()
            if b == 0 and nch > 1:
                ichunk(1, 1).start()
            if b < CH - 1:
                ga(0, b + 1, 1 - p).start()
            elif nch > 1:
                ichunk(1, 1).wait()
                ga(1, 0, 1 - p).start()
            ga(0, b, p).wait()
            sc(0, b, p).start(add=True)

        def body(k, carry):
            q = lax.rem(k, 2)
            qp = 1 - q
            for b in range(CH):
                p = b % 2
                if b == 0:
                    sc(qp, CH - 1, 1 - p).wait()
                else:
                    sc(q, b - 1, 1 - p).wait()
                if b == 0:
                    @pl.when(k + 1 < nch)
                    def _():
                        ichunk(k + 1, qp).start()
                if b < CH - 1:
                    ga(q, b + 1, 1 - p).start()
                else:
                    @pl.when(k + 1 < nch)
                    def _():
                        ichunk(k + 1, qp).wait()
                        ga(qp, 0, 1 - p).start()
                ga(q, b, p).wait()
                sc(q, b, p).start(add=True)
            return carry
        lax.fori_loop(1, nch, body, 0)

        sc((nch - 1) % 2, CH - 1, (CH - 1) % 2).wait()
        plsc.subcore_barrier()
        pltpu.sync_copy(acc_sh.at[pl.ds(s * rpt, rpt)],
                        out_hbm.at[c, pl.ds(s * rpt, rpt)])

    return spmm_kernel


def _dinv_block(degp_blk):
    deg = degp_blk[0, :, 0:1] + degp_blk[1, :, 0:1] + 1.0
    return lax.rsqrt(jnp.maximum(deg, 1.0))


def _tc_a_body(x_ref, w_ref, degp_ref, out_ref):
    dinv = _dinv_block(degp_ref[...])
    h = jnp.dot(x_ref[...], w_ref[...], preferred_element_type=jnp.float32)
    hs = h * dinv
    f = h.shape[1] // 2
    out_ref[0] = hs[:, :f]
    out_ref[1] = hs[:, f:]


def _tc_b_body(acc_ref, degp_ref, b1_ref, w2_ref, out_ref):
    dinv = _dinv_block(degp_ref[...])
    accf = jnp.concatenate([acc_ref[0], acc_ref[1]], axis=1)
    h1 = jnp.maximum(accf * dinv + b1_ref[...], 0.0)
    out_ref[...] = jnp.dot(h1.astype(jnp.bfloat16), w2_ref[...],
                           preferred_element_type=jnp.float32) * dinv


def _tc_c_body(acc_ref, hs2_ref, degp_ref, b2_ref, out_ref):
    dinv = _dinv_block(degp_ref[...])
    accf = acc_ref[0] + acc_ref[1] - hs2_ref[...]
    out_ref[...] = jnp.maximum(accf * dinv + b2_ref[...], 0.0)


def _row_spec(r, width):
    return pl.BlockSpec((r, width), lambda i: (i, 0))


def _half_spec(r, half):
    return pl.BlockSpec((2, r, half), lambda i: (0, i, 0))


def _degp_spec(r):
    return pl.BlockSpec((2, r, 1), lambda i: (0, i, 0))


def _full_spec(shape):
    return pl.BlockSpec(shape, lambda i: (0,) * len(shape))


def kernel(x, edge_index, W1, b1, W2, b2):
    n, d_in = x.shape
    d_hid = W1.shape[1]
    d_out = W2.shape[1]
    e = edge_index.shape[1]

    chunk = NW * B * CH
    e_pad = -(-e // chunk) * chunk
    nch = e_pad // (NS * B * CH)
    nch2 = e_pad // (NW * B * CH)

    src = edge_index[0].astype(jnp.int32)
    dst = edge_index[1].astype(jnp.int32)
    pad = e_pad - e
    npad_rows = N_PAD - n
    dummy = n + jnp.arange(pad, dtype=jnp.int32) % npad_rows
    src_p = jnp.concatenate([src, dummy])
    dst_p = jnp.concatenate([dst, dummy])
    chunks = e_pad // (CH * B)
    sd = jnp.stack([src_p.reshape(chunks, CH, B),
                    dst_p.reshape(chunks, CH, B)], axis=2)
    sd5 = sd.reshape(NS, nch, CH, 2, B)
    sdw5 = sd.reshape(NW, nch2, CH, 2, B)
    nv = e_pad // (NW * 16)
    dst16 = dst_p.reshape(NW, nv, 16)
    iota_nr = jnp.arange(N_PAD // 128, dtype=jnp.int32)

    x_p = jnp.zeros((N_PAD, d_in), jnp.bfloat16).at[:n].set(
        x.astype(jnp.bfloat16))
    w1b = W1.astype(jnp.bfloat16)
    w2b = W2.astype(jnp.bfloat16)
    b1r = b1.reshape(1, d_hid)
    b2r = b2.reshape(1, d_out)

    grid = (N_PAD // TC_ROWS,)
    r = TC_ROWS

    degp = _make_deg_hist(N_PAD, nv)(dst16, iota_nr).reshape(NC, N_PAD, 1)

    hs1 = pl.pallas_call(
        _tc_a_body,
        grid=grid,
        in_specs=[_row_spec(r, d_in), _full_spec((d_in, d_hid)), _degp_spec(r)],
        out_specs=_half_spec(r, d_hid // 2),
        out_shape=jax.ShapeDtypeStruct((2, N_PAD, d_hid // 2), jnp.float32),
    )(x_p, w1b, degp)

    acc1 = _make_spmm(N_PAD, d_hid // 2, nch, False)(hs1, sd5)

    hs2 = pl.pallas_call(
        _tc_b_body,
        grid=grid,
        in_specs=[_half_spec(r, d_hid // 2), _degp_spec(r),
                  _full_spec((1, d_hid)), _full_spec((d_hid, d_out))],
        out_specs=_row_spec(r, d_out),
        out_shape=jax.ShapeDtypeStruct((N_PAD, d_out), jnp.float32),
    )(acc1, degp, b1r, w2b)

    acc2 = _make_spmm(N_PAD, d_out, nch2, True)(hs2, sdw5)

    rc = 2000
    out = pl.pallas_call(
        _tc_c_body,
        grid=(n // rc,),
        in_specs=[_half_spec(rc, d_out), _row_spec(rc, d_out),
                  _degp_spec(rc), _full_spec((1, d_out))],
        out_specs=_row_spec(rc, d_out),
        out_shape=jax.ShapeDtypeStruct((n, d_out), jnp.float32),
    )(acc2, hs2, degp, b2r)

    return out
